# Initial kernel scaffold; baseline (speedup 1.0000x reference)
import functools

import jax
import jax.numpy as jnp
import numpy as np
from jax import lax
from jax.experimental import pallas as pl
from jax.experimental.pallas import tpu as pltpu
from jax.experimental.pallas import tpu_sc as plsc

N = 10000
E = 320000
DIM = 128

NUM_CORES = 2
NUM_SUBCORES = 16
NUM_WORKERS = NUM_CORES * NUM_SUBCORES
EDGES_PER_WORKER = E // NUM_WORKERS
WIN = 80
NUM_WINDOWS = EDGES_PER_WORKER // WIN
ROWS_PER_SUBCORE = N // NUM_SUBCORES
ZROWS = 125


def _filter_body(e_ref, rs_ref, sig_ref, o_ref):
    e = e_ref[...]
    rs = rs_ref[0, 0]
    sig = sig_ref[0, 0]
    g = jnp.exp(-jnp.square(e - rs) / jnp.square(sig))
    w = 0.5 * jnp.cos(np.pi * e)
    o_ref[...] = g * w * (e < 1.0).astype(jnp.float32)


def _edge_filter(e2d, rs, sig):
    return pl.pallas_call(
        _filter_body,
        out_shape=jax.ShapeDtypeStruct(e2d.shape, jnp.float32),
        in_specs=[
            pl.BlockSpec(e2d.shape, lambda: (0, 0)),
            pl.BlockSpec(memory_space=pltpu.SMEM),
            pl.BlockSpec(memory_space=pltpu.SMEM),
        ],
        out_specs=pl.BlockSpec(e2d.shape, lambda: (0, 0)),
    )(e2d, rs, sig)


def _sc_body(h_hbm, f_hbm, src_hbm, dst_hbm, out_hbm,
             src_v, dst_v, f_v, rows_v, zero_v, acc_sh):
    c = lax.axis_index("c")
    s = lax.axis_index("s")
    wid = c * NUM_SUBCORES + s

    @pl.loop(0, ZROWS)
    def _(i):
        for j in range(DIM // 16):
            zero_v[i, pl.ds(j * 16, 16)] = jnp.zeros((16,), jnp.float32)

    for k in range(ROWS_PER_SUBCORE // ZROWS):
        pltpu.sync_copy(
            zero_v,
            acc_sh.at[pl.ds(s * ROWS_PER_SUBCORE + k * ZROWS, ZROWS), :],
        )
    plsc.subcore_barrier()

    base = wid * EDGES_PER_WORKER

    @pl.loop(0, NUM_WINDOWS)
    def _(w):
        off = base + w * WIN
        pltpu.sync_copy(src_hbm.at[pl.ds(off, WIN)], src_v)
        pltpu.sync_copy(dst_hbm.at[pl.ds(off, WIN)], dst_v)
        pltpu.sync_copy(f_hbm.at[pl.ds(off, WIN)], f_v)
        pltpu.sync_copy(h_hbm.at[src_v], rows_v)

        @pl.loop(0, WIN)
        def _(i):
            fv = f_v[i]
            for j in range(DIM // 16):
                sl = pl.ds(j * 16, 16)
                rows_v[i, sl] = rows_v[i, sl] * fv

        pltpu.sync_copy(rows_v, acc_sh.at[dst_v], add=True)

    plsc.subcore_barrier()

    pltpu.sync_copy(
        acc_sh.at[pl.ds(s * ROWS_PER_SUBCORE, ROWS_PER_SUBCORE), :],
        out_hbm.at[c, pl.ds(s * ROWS_PER_SUBCORE, ROWS_PER_SUBCORE), :],
    )


def _sc_scatter(h, f, src, dst):
    mesh = plsc.VectorSubcoreMesh(core_axis_name="c", subcore_axis_name="s")
    kern = pl.kernel(
        _sc_body,
        out_type=jax.ShapeDtypeStruct((NUM_CORES, N, DIM), jnp.float32),
        mesh=mesh,
        scratch_types=[
            pltpu.VMEM((WIN,), jnp.int32),
            pltpu.VMEM((WIN,), jnp.int32),
            pltpu.VMEM((WIN,), jnp.float32),
            pltpu.VMEM((WIN, DIM), jnp.float32),
            pltpu.VMEM((ZROWS, DIM), jnp.float32),
            pltpu.VMEM_SHARED((N, DIM), jnp.float32),
        ],
    )
    return kern(h, f, src, dst)


ROW_BLK = 1000


def _linear_body(p_ref, v_ref, wt_ref, b_ref, o_ref):
    x = p_ref[0] + p_ref[1] + v_ref[...]
    y = jnp.dot(x, wt_ref[...], preferred_element_type=jnp.float32)
    o_ref[...] = jnp.maximum(y + b_ref[...], 0.0)


def _linear_relu(p, v, wt, b2d):
    grid = (N // ROW_BLK,)
    return pl.pallas_call(
        _linear_body,
        grid=grid,
        out_shape=jax.ShapeDtypeStruct((N, DIM), jnp.float32),
        in_specs=[
            pl.BlockSpec((NUM_CORES, ROW_BLK, DIM), lambda i: (0, i, 0)),
            pl.BlockSpec((ROW_BLK, DIM), lambda i: (i, 0)),
            pl.BlockSpec((DIM, DIM), lambda i: (0, 0)),
            pl.BlockSpec((1, DIM), lambda i: (0, 0)),
        ],
        out_specs=pl.BlockSpec((ROW_BLK, DIM), lambda i: (i, 0)),
    )(p, v, wt, b2d)


def kernel(v, e, rs, sigma, W, b, edge_index):
    src = edge_index[0]
    dst = edge_index[1]
    f2d = _edge_filter(
        e.reshape(E // DIM, DIM),
        rs.reshape(1, 1),
        sigma.reshape(1, 1),
    )
    f = f2d.reshape(E)
    wt = W.T
    b2d = b.reshape(1, DIM)

    h = v
    for _ in range(3):
        p = _sc_scatter(h, f, src, dst)
        h = _linear_relu(p, v, wt, b2d)
    return h


# speedup vs baseline: 3.7513x; 3.7513x over previous
import functools

import jax
import jax.numpy as jnp
import numpy as np
from jax import lax
from jax.experimental import pallas as pl
from jax.experimental.pallas import tpu as pltpu
from jax.experimental.pallas import tpu_sc as plsc

N = 10000
E = 320000
DIM = 128

NUM_CORES = 2
NUM_SUBCORES = 16
NUM_WORKERS = NUM_CORES * NUM_SUBCORES
EDGES_PER_WORKER = E // NUM_WORKERS
WIN = 80
NUM_WINDOWS = EDGES_PER_WORKER // WIN
ROWS_PER_SUBCORE = 624
ZROWS = 312


def _filter_body(e_ref, rs_ref, sig_ref, o_ref):
    e = e_ref[...]
    rs = rs_ref[0, 0]
    sig = sig_ref[0, 0]
    g = jnp.exp(-jnp.square(e - rs) / jnp.square(sig))
    w = 0.5 * jnp.cos(np.pi * e)
    o_ref[...] = g * w * (e < 1.0).astype(jnp.float32)


def _edge_filter(e2d, rs, sig):
    return pl.pallas_call(
        _filter_body,
        out_shape=jax.ShapeDtypeStruct(e2d.shape, jnp.float32),
        in_specs=[
            pl.BlockSpec(e2d.shape, lambda: (0, 0)),
            pl.BlockSpec(memory_space=pltpu.SMEM),
            pl.BlockSpec(memory_space=pltpu.SMEM),
        ],
        out_specs=pl.BlockSpec(e2d.shape, lambda: (0, 0)),
    )(e2d, rs, sig)


def _sc_body(h_hbm, f_hbm, src_hbm, dst_hbm, out_hbm,
             src_v, dst_v, f_v, rows_v, zero_v, acc_sh):
    c = lax.axis_index("c")
    s = lax.axis_index("s")
    wid = c * NUM_SUBCORES + s

    @pl.loop(0, ZROWS)
    def _(i):
        for j in range(DIM // 16):
            zero_v[i, pl.ds(j * 16, 16)] = jnp.zeros((16,), jnp.float32)

    for k in range(ROWS_PER_SUBCORE // ZROWS):
        pltpu.sync_copy(
            zero_v,
            acc_sh.at[pl.ds(s * ROWS_PER_SUBCORE + k * ZROWS, ZROWS), :],
        )

    @pl.when(s == NUM_SUBCORES - 1)
    def _():
        pltpu.sync_copy(
            zero_v.at[pl.ds(0, 16), :],
            acc_sh.at[pl.ds(NUM_SUBCORES * ROWS_PER_SUBCORE, 16), :],
        )

    plsc.subcore_barrier()

    base = wid * EDGES_PER_WORKER

    @pl.loop(0, NUM_WINDOWS)
    def _(w):
        off = base + w * WIN
        pltpu.sync_copy(src_hbm.at[pl.ds(off, WIN)], src_v)
        pltpu.sync_copy(dst_hbm.at[pl.ds(off, WIN)], dst_v)
        pltpu.sync_copy(f_hbm.at[pl.ds(off, WIN)], f_v)
        pltpu.sync_copy(h_hbm.at[src_v], rows_v)

        @pl.loop(0, WIN // 16)
        def _(g):
            fvec = f_v[pl.ds(g * 16, 16)]
            for l in range(16):
                fv = fvec[l]
                row = g * 16 + l
                for j in range(DIM // 16):
                    sl = pl.ds(j * 16, 16)
                    rows_v[row, sl] = rows_v[row, sl] * fv

        pltpu.sync_copy(rows_v, acc_sh.at[dst_v], add=True)

    plsc.subcore_barrier()

    pltpu.sync_copy(
        acc_sh.at[pl.ds(s * ROWS_PER_SUBCORE, ROWS_PER_SUBCORE), :],
        out_hbm.at[c, pl.ds(s * ROWS_PER_SUBCORE, ROWS_PER_SUBCORE), :],
    )

    @pl.when(s == NUM_SUBCORES - 1)
    def _():
        pltpu.sync_copy(
            acc_sh.at[pl.ds(NUM_SUBCORES * ROWS_PER_SUBCORE, 16), :],
            out_hbm.at[c, pl.ds(NUM_SUBCORES * ROWS_PER_SUBCORE, 16), :],
        )


def _sc_scatter(h, f, src, dst):
    mesh = plsc.VectorSubcoreMesh(core_axis_name="c", subcore_axis_name="s")
    kern = pl.kernel(
        _sc_body,
        out_type=jax.ShapeDtypeStruct((NUM_CORES, N, DIM), jnp.float32),
        mesh=mesh,
        scratch_types=[
            pltpu.VMEM((WIN,), jnp.int32),
            pltpu.VMEM((WIN,), jnp.int32),
            pltpu.VMEM((WIN,), jnp.float32),
            pltpu.VMEM((WIN, DIM), jnp.float32),
            pltpu.VMEM((ZROWS, DIM), jnp.float32),
            pltpu.VMEM_SHARED((N, DIM), jnp.float32),
        ],
    )
    return kern(h, f, src, dst)


ROW_BLK = 1000


def _linear_body(p_ref, v_ref, wt_ref, b_ref, o_ref):
    x = p_ref[0] + p_ref[1] + v_ref[...]
    y = jnp.dot(x, wt_ref[...], preferred_element_type=jnp.float32)
    o_ref[...] = jnp.maximum(y + b_ref[...], 0.0)


def _linear_relu(p, v, wt, b2d):
    grid = (N // ROW_BLK,)
    return pl.pallas_call(
        _linear_body,
        grid=grid,
        out_shape=jax.ShapeDtypeStruct((N, DIM), jnp.float32),
        in_specs=[
            pl.BlockSpec((NUM_CORES, ROW_BLK, DIM), lambda i: (0, i, 0)),
            pl.BlockSpec((ROW_BLK, DIM), lambda i: (i, 0)),
            pl.BlockSpec((DIM, DIM), lambda i: (0, 0)),
            pl.BlockSpec((1, DIM), lambda i: (0, 0)),
        ],
        out_specs=pl.BlockSpec((ROW_BLK, DIM), lambda i: (i, 0)),
    )(p, v, wt, b2d)


def kernel(v, e, rs, sigma, W, b, edge_index):
    src = edge_index[0]
    dst = edge_index[1]
    f2d = _edge_filter(
        e.reshape(E // DIM, DIM),
        rs.reshape(1, 1),
        sigma.reshape(1, 1),
    )
    f = f2d.reshape(E)
    wt = W.T
    b2d = b.reshape(1, DIM)

    h = v
    for _ in range(3):
        p = _sc_scatter(h, f, src, dst)
        h = _linear_relu(p, v, wt, b2d)
    return h


# speedup vs baseline: 5.3303x; 1.4209x over previous
import functools

import jax
import jax.numpy as jnp
import numpy as np
from jax import lax
from jax.experimental import pallas as pl
from jax.experimental.pallas import tpu as pltpu
from jax.experimental.pallas import tpu_sc as plsc

N = 10000
E = 320000
DIM = 128

NUM_CORES = 2
NUM_SUBCORES = 16
NUM_WORKERS = NUM_CORES * NUM_SUBCORES
EDGES_PER_WORKER = E // NUM_WORKERS
WIN = 80
NUM_WINDOWS = EDGES_PER_WORKER // WIN
ROWS_PER_SUBCORE = 624
ZROWS = 312


def _filter_body(e_ref, rs_ref, sig_ref, o_ref):
    e = e_ref[...]
    rs = rs_ref[0, 0]
    sig = sig_ref[0, 0]
    g = jnp.exp(-jnp.square(e - rs) / jnp.square(sig))
    w = 0.5 * jnp.cos(np.pi * e)
    o_ref[...] = g * w * (e < 1.0).astype(jnp.float32)


def _edge_filter(e2d, rs, sig):
    return pl.pallas_call(
        _filter_body,
        out_shape=jax.ShapeDtypeStruct(e2d.shape, jnp.float32),
        in_specs=[
            pl.BlockSpec(e2d.shape, lambda: (0, 0)),
            pl.BlockSpec(memory_space=pltpu.SMEM),
            pl.BlockSpec(memory_space=pltpu.SMEM),
        ],
        out_specs=pl.BlockSpec(e2d.shape, lambda: (0, 0)),
    )(e2d, rs, sig)


def _sc_body(h_hbm, f_hbm, src_hbm, dst_hbm, out_hbm,
             src0, dst0, f0, rows0, src1, dst1, f1, rows1,
             acc_sh, sem0, sem1):
    c = lax.axis_index("c")
    s = lax.axis_index("s")
    wid = c * NUM_SUBCORES + s

    @pl.loop(0, WIN)
    def _(i):
        for j in range(DIM // 16):
            rows0[i, pl.ds(j * 16, 16)] = jnp.zeros((16,), jnp.float32)

    for k in range(ROWS_PER_SUBCORE // WIN):
        pltpu.sync_copy(
            rows0,
            acc_sh.at[pl.ds(s * ROWS_PER_SUBCORE + k * WIN, WIN), :],
        )
    pltpu.sync_copy(
        rows0.at[pl.ds(0, 64), :],
        acc_sh.at[pl.ds(s * ROWS_PER_SUBCORE + 560, 64), :],
    )

    @pl.when(s == NUM_SUBCORES - 1)
    def _():
        pltpu.sync_copy(
            rows0.at[pl.ds(0, 16), :],
            acc_sh.at[pl.ds(NUM_SUBCORES * ROWS_PER_SUBCORE, 16), :],
        )

    plsc.subcore_barrier()

    base = wid * EDGES_PER_WORKER

    def prep(w, src_v, dst_v, f_v, rows_v, sem):
        off = base + w * WIN
        pltpu.sync_copy(src_hbm.at[pl.ds(off, WIN)], src_v)
        pltpu.sync_copy(dst_hbm.at[pl.ds(off, WIN)], dst_v)
        pltpu.sync_copy(f_hbm.at[pl.ds(off, WIN)], f_v)
        pltpu.async_copy(h_hbm.at[src_v], rows_v, sem)

    def process(src_v, dst_v, f_v, rows_v, sem):
        pltpu.make_async_copy(h_hbm.at[src_v], rows_v, sem).wait()

        @pl.loop(0, WIN // 16)
        def _(g):
            fvec = f_v[pl.ds(g * 16, 16)]
            for l in range(16):
                fv = fvec[l]
                row = g * 16 + l
                for j in range(DIM // 16):
                    sl = pl.ds(j * 16, 16)
                    rows_v[row, sl] = rows_v[row, sl] * fv

        pltpu.sync_copy(rows_v, acc_sh.at[dst_v], add=True)

    prep(0, src0, dst0, f0, rows0, sem0)
    prep(1, src1, dst1, f1, rows1, sem1)

    @pl.loop(0, (NUM_WINDOWS - 1) // 2)
    def _(p):
        w0 = 2 * p
        process(src0, dst0, f0, rows0, sem0)
        prep(w0 + 2, src0, dst0, f0, rows0, sem0)
        process(src1, dst1, f1, rows1, sem1)

        @pl.when(w0 + 3 < NUM_WINDOWS)
        def _():
            prep(w0 + 3, src1, dst1, f1, rows1, sem1)

    process(src0, dst0, f0, rows0, sem0)

    plsc.subcore_barrier()

    pltpu.sync_copy(
        acc_sh.at[pl.ds(s * ROWS_PER_SUBCORE, ROWS_PER_SUBCORE), :],
        out_hbm.at[c, pl.ds(s * ROWS_PER_SUBCORE, ROWS_PER_SUBCORE), :],
    )

    @pl.when(s == NUM_SUBCORES - 1)
    def _():
        pltpu.sync_copy(
            acc_sh.at[pl.ds(NUM_SUBCORES * ROWS_PER_SUBCORE, 16), :],
            out_hbm.at[c, pl.ds(NUM_SUBCORES * ROWS_PER_SUBCORE, 16), :],
        )


def _sc_scatter(h, f, src, dst):
    mesh = plsc.VectorSubcoreMesh(core_axis_name="c", subcore_axis_name="s")
    kern = pl.kernel(
        _sc_body,
        out_type=jax.ShapeDtypeStruct((NUM_CORES, N, DIM), jnp.float32),
        mesh=mesh,
        scratch_types=[
            pltpu.VMEM((WIN,), jnp.int32),
            pltpu.VMEM((WIN,), jnp.int32),
            pltpu.VMEM((WIN,), jnp.float32),
            pltpu.VMEM((WIN, DIM), jnp.float32),
            pltpu.VMEM((WIN,), jnp.int32),
            pltpu.VMEM((WIN,), jnp.int32),
            pltpu.VMEM((WIN,), jnp.float32),
            pltpu.VMEM((WIN, DIM), jnp.float32),
            pltpu.VMEM_SHARED((N, DIM), jnp.float32),
            pltpu.SemaphoreType.DMA,
            pltpu.SemaphoreType.DMA,
        ],
    )
    return kern(h, f, src, dst)


ROW_BLK = 1000


def _linear_body(p_ref, v_ref, wt_ref, b_ref, o_ref):
    x = p_ref[0] + p_ref[1] + v_ref[...]
    y = jnp.dot(x, wt_ref[...], preferred_element_type=jnp.float32)
    o_ref[...] = jnp.maximum(y + b_ref[...], 0.0)


def _linear_relu(p, v, wt, b2d):
    grid = (N // ROW_BLK,)
    return pl.pallas_call(
        _linear_body,
        grid=grid,
        out_shape=jax.ShapeDtypeStruct((N, DIM), jnp.float32),
        in_specs=[
            pl.BlockSpec((NUM_CORES, ROW_BLK, DIM), lambda i: (0, i, 0)),
            pl.BlockSpec((ROW_BLK, DIM), lambda i: (i, 0)),
            pl.BlockSpec((DIM, DIM), lambda i: (0, 0)),
            pl.BlockSpec((1, DIM), lambda i: (0, 0)),
        ],
        out_specs=pl.BlockSpec((ROW_BLK, DIM), lambda i: (i, 0)),
    )(p, v, wt, b2d)


def kernel(v, e, rs, sigma, W, b, edge_index):
    src = edge_index[0]
    dst = edge_index[1]
    f2d = _edge_filter(
        e.reshape(E // DIM, DIM),
        rs.reshape(1, 1),
        sigma.reshape(1, 1),
    )
    f = f2d.reshape(E)
    wt = W.T
    b2d = b.reshape(1, DIM)

    h = v
    for _ in range(3):
        p = _sc_scatter(h, f, src, dst)
        h = _linear_relu(p, v, wt, b2d)
    return h
